# baseline (speedup 1.0000x reference)
import functools

import jax
import jax.numpy as jnp
from jax import lax
from jax.experimental import pallas as pl
from jax.experimental.pallas import tpu as pltpu
from jax.experimental.pallas import tpu_sc as plsc

N = 100000
E = 200000
NC = 2
NS = 16
NW = NC * NS

LINK_CHUNK = 6272
NODE_CHUNK = 3136


@functools.cache
def _mesh():
    return plsc.VectorSubcoreMesh(core_axis_name="c", subcore_axis_name="s",
                                  num_cores=NC, num_subcores=NS)


def _wid():
    return lax.axis_index("s") * NC + lax.axis_index("c")


def _signed_body(disch, status, head, tail, over, signed_out,
                 table_v, d_v, s_v, h_v, t_v, sem):
    base = jnp.minimum(_wid() * LINK_CHUNK, E - LINK_CHUNK)
    base = pl.multiple_of(base, 8)
    sl = pl.ds(base, LINK_CHUNK)
    cp_t = pltpu.async_copy(over, table_v, sem)
    cp_d = pltpu.async_copy(disch.at[sl], d_v, sem)
    cp_s = pltpu.async_copy(status.at[sl], s_v, sem)
    cp_h = pltpu.async_copy(head.at[sl], h_v, sem)
    cp_u = pltpu.async_copy(tail.at[sl], t_v, sem)
    cp_t.wait()
    cp_d.wait()
    cp_s.wait()
    cp_h.wait()
    cp_u.wait()

    def body(i, carry):
        v = pl.ds(i * 16, 16)
        oh = plsc.load_gather(table_v, [h_v[v]])
        ot = plsc.load_gather(table_v, [t_v[v]])
        d = jnp.where(s_v[v] == 4, 0.0, d_v[v])
        d_v[v] = jnp.where(oh > ot, d, -d)
        return carry

    lax.fori_loop(0, LINK_CHUNK // 16, body, 0)
    pltpu.sync_copy(d_v, signed_out.at[sl])


def _flux_body(links, signed, melt, out, idx_v, g_v, m_v, o_v, sem):
    base = jnp.minimum(_wid() * NODE_CHUNK, N - NODE_CHUNK)
    base = pl.multiple_of(base, 8)
    lbase = pl.multiple_of(base * 4, 8)
    cp_i = pltpu.async_copy(links.at[pl.ds(lbase, NODE_CHUNK * 4)], idx_v, sem)
    cp_m = pltpu.async_copy(melt.at[pl.ds(base, NODE_CHUNK)], m_v, sem)
    cp_i.wait()
    pltpu.async_copy(signed.at[idx_v], g_v, sem).wait()
    cp_m.wait()

    lane = lax.iota(jnp.int32, 16)

    def body(j, carry):
        nb = j * 16
        i0 = (nb + lane) * 4
        acc = plsc.load_gather(g_v, [i0])
        acc = acc + plsc.load_gather(g_v, [i0 + 1])
        acc = acc + plsc.load_gather(g_v, [i0 + 2])
        acc = acc + plsc.load_gather(g_v, [i0 + 3])
        v = pl.ds(nb, 16)
        o_v[v] = acc - m_v[v]
        return carry

    lax.fori_loop(0, NODE_CHUNK // 16, body, 0)
    pltpu.sync_copy(o_v, out.at[pl.ds(base, NODE_CHUNK)])


@functools.cache
def _signed_call():
    return pl.kernel(
        _signed_body,
        out_type=jax.ShapeDtypeStruct((E,), jnp.float32),
        mesh=_mesh(),
        compiler_params=pltpu.CompilerParams(needs_layout_passes=False),
        scratch_types=[
            pltpu.VMEM((N,), jnp.float32),
            pltpu.VMEM((LINK_CHUNK,), jnp.float32),
            pltpu.VMEM((LINK_CHUNK,), jnp.int32),
            pltpu.VMEM((LINK_CHUNK,), jnp.int32),
            pltpu.VMEM((LINK_CHUNK,), jnp.int32),
            pltpu.SemaphoreType.DMA,
        ],
    )


@functools.cache
def _flux_call():
    return pl.kernel(
        _flux_body,
        out_type=jax.ShapeDtypeStruct((N,), jnp.float32),
        mesh=_mesh(),
        compiler_params=pltpu.CompilerParams(needs_layout_passes=False),
        scratch_types=[
            pltpu.VMEM((NODE_CHUNK * 4,), jnp.int32),
            pltpu.VMEM((NODE_CHUNK * 4,), jnp.float32),
            pltpu.VMEM((NODE_CHUNK,), jnp.float32),
            pltpu.VMEM((NODE_CHUNK,), jnp.float32),
            pltpu.SemaphoreType.DMA,
        ],
    )


def kernel(discharge, overburden, melt_rate, status_at_link,
           node_at_link_head, node_at_link_tail, links_at_node):
    status = status_at_link.astype(jnp.int32)
    head = node_at_link_head.astype(jnp.int32)
    tail = node_at_link_tail.astype(jnp.int32)
    links = links_at_node.astype(jnp.int32).reshape(N * 4)
    signed = _signed_call()(discharge, status, head, tail, overburden)
    return _flux_call()(links, signed, melt_rate)


# speedup vs baseline: 1.0082x; 1.0082x over previous
import functools

import jax
import jax.numpy as jnp
from jax import lax
from jax.experimental import pallas as pl
from jax.experimental.pallas import tpu as pltpu
from jax.experimental.pallas import tpu_sc as plsc

N = 100000
E = 200000
NC = 2
NS = 16
NW = NC * NS

LINK_CHUNK = 6272
NODE_CHUNK = 3136


@functools.cache
def _mesh():
    return plsc.VectorSubcoreMesh(core_axis_name="c", subcore_axis_name="s",
                                  num_cores=NC, num_subcores=NS)


def _wid():
    return lax.axis_index("s") * NC + lax.axis_index("c")


_TCH = 6248
_TTAIL = N - 16 * _TCH


def _signed_body(disch, status, head, tail, over, signed_out,
                 table_v, d_v, s_v, h_v, t_v, sem):
    base = jnp.minimum(_wid() * LINK_CHUNK, E - LINK_CHUNK)
    base = pl.multiple_of(base, 8)
    sl = pl.ds(base, LINK_CHUNK)
    rot = (lax.axis_index("s") + 8 * lax.axis_index("c")) % 16
    cps = []
    for k in range(16):
        c = lax.rem(rot + k, 16)
        off = pl.multiple_of(c * _TCH, 8)
        cps.append(pltpu.async_copy(over.at[pl.ds(off, _TCH)],
                                    table_v.at[pl.ds(off, _TCH)], sem))
    cps.append(pltpu.async_copy(over.at[pl.ds(16 * _TCH, _TTAIL)],
                                table_v.at[pl.ds(16 * _TCH, _TTAIL)], sem))
    cps.append(pltpu.async_copy(disch.at[sl], d_v, sem))
    cps.append(pltpu.async_copy(status.at[sl], s_v, sem))
    cps.append(pltpu.async_copy(head.at[sl], h_v, sem))
    cps.append(pltpu.async_copy(tail.at[sl], t_v, sem))
    for cp in cps:
        cp.wait()

    def body(i, carry):
        v = pl.ds(i * 16, 16)
        oh = plsc.load_gather(table_v, [h_v[v]])
        ot = plsc.load_gather(table_v, [t_v[v]])
        d = jnp.where(s_v[v] == 4, 0.0, d_v[v])
        d_v[v] = jnp.where(oh > ot, d, -d)
        return carry

    lax.fori_loop(0, LINK_CHUNK // 16, body, 0)
    pltpu.sync_copy(d_v, signed_out.at[sl])


def _flux_body(links, signed, melt, out, idx_v, g_v, m_v, o_v, sem):
    base = jnp.minimum(_wid() * NODE_CHUNK, N - NODE_CHUNK)
    base = pl.multiple_of(base, 8)
    lbase = pl.multiple_of(base * 4, 8)
    cp_i = pltpu.async_copy(links.at[pl.ds(lbase, NODE_CHUNK * 4)], idx_v, sem)
    cp_m = pltpu.async_copy(melt.at[pl.ds(base, NODE_CHUNK)], m_v, sem)
    cp_i.wait()
    pltpu.async_copy(signed.at[idx_v], g_v, sem).wait()
    cp_m.wait()

    lane = lax.iota(jnp.int32, 16)

    def body(j, carry):
        nb = j * 16
        i0 = (nb + lane) * 4
        acc = plsc.load_gather(g_v, [i0])
        acc = acc + plsc.load_gather(g_v, [i0 + 1])
        acc = acc + plsc.load_gather(g_v, [i0 + 2])
        acc = acc + plsc.load_gather(g_v, [i0 + 3])
        v = pl.ds(nb, 16)
        o_v[v] = acc - m_v[v]
        return carry

    lax.fori_loop(0, NODE_CHUNK // 16, body, 0)
    pltpu.sync_copy(o_v, out.at[pl.ds(base, NODE_CHUNK)])


@functools.cache
def _signed_call():
    return pl.kernel(
        _signed_body,
        out_type=jax.ShapeDtypeStruct((E,), jnp.float32),
        mesh=_mesh(),
        compiler_params=pltpu.CompilerParams(needs_layout_passes=False),
        scratch_types=[
            pltpu.VMEM((N,), jnp.float32),
            pltpu.VMEM((LINK_CHUNK,), jnp.float32),
            pltpu.VMEM((LINK_CHUNK,), jnp.int32),
            pltpu.VMEM((LINK_CHUNK,), jnp.int32),
            pltpu.VMEM((LINK_CHUNK,), jnp.int32),
            pltpu.SemaphoreType.DMA,
        ],
    )


@functools.cache
def _flux_call():
    return pl.kernel(
        _flux_body,
        out_type=jax.ShapeDtypeStruct((N,), jnp.float32),
        mesh=_mesh(),
        compiler_params=pltpu.CompilerParams(needs_layout_passes=False),
        scratch_types=[
            pltpu.VMEM((NODE_CHUNK * 4,), jnp.int32),
            pltpu.VMEM((NODE_CHUNK * 4,), jnp.float32),
            pltpu.VMEM((NODE_CHUNK,), jnp.float32),
            pltpu.VMEM((NODE_CHUNK,), jnp.float32),
            pltpu.SemaphoreType.DMA,
        ],
    )


def kernel(discharge, overburden, melt_rate, status_at_link,
           node_at_link_head, node_at_link_tail, links_at_node):
    status = status_at_link.astype(jnp.int32)
    head = node_at_link_head.astype(jnp.int32)
    tail = node_at_link_tail.astype(jnp.int32)
    links = links_at_node.astype(jnp.int32).reshape(N * 4)
    signed = _signed_call()(discharge, status, head, tail, overburden)
    return _flux_call()(links, signed, melt_rate)


# speedup vs baseline: 1.6845x; 1.6708x over previous
import functools

import jax
import jax.numpy as jnp
from jax import lax
from jax.experimental import pallas as pl
from jax.experimental.pallas import tpu as pltpu
from jax.experimental.pallas import tpu_sc as plsc

N = 100000
E = 200000
NC = 2
NS = 16
NW = NC * NS

LINK_CHUNK = 6272
NSUB = 4
SUBL = LINK_CHUNK // NSUB
NODE_CHUNK = 3136


@functools.cache
def _mesh():
    return plsc.VectorSubcoreMesh(core_axis_name="c", subcore_axis_name="s",
                                  num_cores=NC, num_subcores=NS)


def _wid():
    return lax.axis_index("s") * NC + lax.axis_index("c")


def _signed_body(disch, status, head, tail, over, signed_out,
                 d_v, s_v, h_v, t_v, oh_v, ot_v, o_v,
                 sem_in, g0, g1, sem_out):
    base = jnp.minimum(_wid() * LINK_CHUNK, E - LINK_CHUNK)
    base = pl.multiple_of(base, 8)
    sl = pl.ds(base, LINK_CHUNK)
    cps = [pltpu.async_copy(disch.at[sl], d_v, sem_in),
           pltpu.async_copy(status.at[sl], s_v, sem_in),
           pltpu.async_copy(head.at[sl], h_v, sem_in),
           pltpu.async_copy(tail.at[sl], t_v, sem_in)]
    for cp in cps:
        cp.wait()

    gsem = [g0, g1]
    def issue(c):
        buf = (c % 2) * SUBL
        hh = h_v.at[pl.ds(c * SUBL, SUBL)]
        tt = t_v.at[pl.ds(c * SUBL, SUBL)]
        cg1 = pltpu.async_copy(over.at[hh], oh_v.at[pl.ds(buf, SUBL)],
                               gsem[c % 2])
        cg2 = pltpu.async_copy(over.at[tt], ot_v.at[pl.ds(buf, SUBL)],
                               gsem[c % 2])
        return cg1, cg2

    pend = [issue(0), issue(1)]
    wb = []
    for c in range(NSUB):
        cg1, cg2 = pend[c % 2]
        cg1.wait()
        cg2.wait()
        if c + 2 < NSUB:
            pend[c % 2] = issue(c + 2)
        buf = (c % 2) * SUBL

        def body(i, carry, c=c, buf=buf):
            v = pl.ds(c * SUBL + i * 16, 16)
            vb = pl.ds(buf + i * 16, 16)
            d = jnp.where(s_v[v] == 4, 0.0, d_v[v])
            o_v[v] = jnp.where(oh_v[vb] > ot_v[vb], d, -d)
            return carry

        lax.fori_loop(0, SUBL // 16, body, 0)
        wb.append(pltpu.async_copy(
            o_v.at[pl.ds(c * SUBL, SUBL)],
            signed_out.at[pl.ds(base + c * SUBL, SUBL)], sem_out))
    for cp in wb:
        cp.wait()


def _flux_body(links_t, signed, melt, out, idx_v, g_v, m_v, o_v,
               sem_in, g0, g1, sem_out):
    base = jnp.minimum(_wid() * NODE_CHUNK, N - NODE_CHUNK)
    base = pl.multiple_of(base, 8)
    icp = []
    for l in range(4):
        icp.append(pltpu.async_copy(
            links_t.at[pl.ds(l * N + base, NODE_CHUNK)],
            idx_v.at[pl.ds(l * NODE_CHUNK, NODE_CHUNK)], sem_in))
    mcp = pltpu.async_copy(melt.at[pl.ds(base, NODE_CHUNK)], m_v, sem_in)

    gsem = [g0, g1]

    def issue(l):
        idx = idx_v.at[pl.ds(l * NODE_CHUNK, NODE_CHUNK)]
        dst = g_v.at[pl.ds(l * NODE_CHUNK, NODE_CHUNK)]
        return pltpu.async_copy(signed.at[idx], dst, gsem[l % 2])

    icp[0].wait()
    pend0 = issue(0)
    icp[1].wait()
    pend1 = issue(1)
    icp[2].wait()
    icp[3].wait()
    pend = [pend0, pend1]

    for l in range(4):
        pend[l % 2].wait()
        if l + 2 < 4:
            pend[l % 2] = issue(l + 2)
        if l == 3:
            mcp.wait()
        gb = l * NODE_CHUNK

        def body(j, carry, l=l, gb=gb):
            v = pl.ds(j * 16, 16)
            g = g_v[pl.ds(gb + j * 16, 16)]
            if l == 0:
                o_v[v] = g
            elif l == 3:
                o_v[v] = o_v[v] + g - m_v[v]
            else:
                o_v[v] = o_v[v] + g
            return carry

        lax.fori_loop(0, NODE_CHUNK // 16, body, 0)

    pltpu.sync_copy(o_v, out.at[pl.ds(base, NODE_CHUNK)])


@functools.cache
def _signed_call():
    return pl.kernel(
        _signed_body,
        out_type=jax.ShapeDtypeStruct((E,), jnp.float32),
        mesh=_mesh(),
        scratch_types=[
            pltpu.VMEM((LINK_CHUNK,), jnp.float32),
            pltpu.VMEM((LINK_CHUNK,), jnp.int32),
            pltpu.VMEM((LINK_CHUNK,), jnp.int32),
            pltpu.VMEM((LINK_CHUNK,), jnp.int32),
            pltpu.VMEM((2 * SUBL,), jnp.float32),
            pltpu.VMEM((2 * SUBL,), jnp.float32),
            pltpu.VMEM((LINK_CHUNK,), jnp.float32),
            pltpu.SemaphoreType.DMA,
            pltpu.SemaphoreType.DMA,
            pltpu.SemaphoreType.DMA,
            pltpu.SemaphoreType.DMA,
        ],
    )


@functools.cache
def _flux_call():
    return pl.kernel(
        _flux_body,
        out_type=jax.ShapeDtypeStruct((N,), jnp.float32),
        mesh=_mesh(),
        scratch_types=[
            pltpu.VMEM((NODE_CHUNK * 4,), jnp.int32),
            pltpu.VMEM((NODE_CHUNK * 4,), jnp.float32),
            pltpu.VMEM((NODE_CHUNK,), jnp.float32),
            pltpu.VMEM((NODE_CHUNK,), jnp.float32),
            pltpu.SemaphoreType.DMA,
            pltpu.SemaphoreType.DMA,
            pltpu.SemaphoreType.DMA,
            pltpu.SemaphoreType.DMA,
        ],
    )


def kernel(discharge, overburden, melt_rate, status_at_link,
           node_at_link_head, node_at_link_tail, links_at_node):
    status = status_at_link.astype(jnp.int32)
    head = node_at_link_head.astype(jnp.int32)
    tail = node_at_link_tail.astype(jnp.int32)
    links_t = links_at_node.astype(jnp.int32).T.reshape(4 * N)
    signed = _signed_call()(discharge, status, head, tail, overburden)
    return _flux_call()(links_t, signed, melt_rate)
